# baseline (speedup 1.0000x reference)
import functools

import jax
import jax.numpy as jnp
from jax import lax
from jax.experimental import pallas as pl
from jax.experimental.pallas import tpu as pltpu
from jax.experimental.pallas import tpu_sc as plsc

N = 10000
NP = 10240
E = 320000
ERV = E // 128
F = 128
FH = 64
C16 = 16
BM = 10000

_mesh = plsc.VectorSubcoreMesh(core_axis_name="c", subcore_axis_name="s")
_sc_params = pltpu.CompilerParams(use_tc_tiling_on_sc=False)


@functools.partial(
    pl.kernel,
    out_type=[jax.ShapeDtypeStruct((NP,), jnp.float32),
              jax.ShapeDtypeStruct((NP,), jnp.float32)],
    mesh=_mesh,
    compiler_params=_sc_params,
    scratch_types=[
        pltpu.VMEM((80, 128), jnp.int32),
        pltpu.VMEM((128,), jnp.float32),
        pltpu.VMEM_SHARED((NP,), jnp.float32),
        pltpu.SemaphoreType.DMA,
    ],
)
def _deg_kernel(e3_hbm, zeros_hbm, out0_hbm, out1_hbm, dst_v, ones_v, acc_sh, sem):
    c = lax.axis_index("c")
    s = lax.axis_index("s")
    wid = s * 2 + c
    rows = jnp.where(wid < 31, 80, 20)
    pltpu.sync_copy(zeros_hbm.at[pl.ds(s * 640, 640)], acc_sh.at[pl.ds(s * 640, 640)])
    for k in range(8):
        ones_v[pl.ds(k * 16, 16)] = jnp.full((16,), 1.0, jnp.float32)
    dst_h = e3_hbm.at[1]

    @pl.when(wid < 31)
    def _stage_full():
        pltpu.sync_copy(dst_h.at[pl.ds(wid * 80, 80)], dst_v)

    @pl.when(wid == 31)
    def _stage_tail():
        pltpu.sync_copy(dst_h.at[pl.ds(2480, 20)], dst_v.at[pl.ds(0, 20)])

    plsc.subcore_barrier()

    def fire(j, carry):
        pltpu.async_copy(ones_v, acc_sh.at[dst_v.at[j]], sem, add=True)
        return carry

    lax.fori_loop(0, rows, fire, 0)

    def drain(j, carry):
        pltpu.make_async_copy(ones_v, acc_sh.at[dst_v.at[0]], sem).wait()
        return carry

    lax.fori_loop(0, rows, drain, 0)
    plsc.subcore_barrier()

    @pl.when(c == 0)
    def _out0():
        pltpu.sync_copy(acc_sh.at[pl.ds(s * 640, 640)], out0_hbm.at[pl.ds(s * 640, 640)])

    @pl.when(c == 1)
    def _out1():
        pltpu.sync_copy(acc_sh.at[pl.ds(s * 640, 640)], out1_hbm.at[pl.ds(s * 640, 640)])


@functools.partial(
    pl.kernel,
    out_type=jax.ShapeDtypeStruct((2, NP, FH), jnp.float32),
    mesh=_mesh,
    compiler_params=_sc_params,
    scratch_types=[
        pltpu.VMEM((160, 128), jnp.int32),
        pltpu.VMEM((160, 128), jnp.int32),
        pltpu.VMEM((5, 128, FH), jnp.float32),
        pltpu.VMEM_SHARED((NP, FH), jnp.float32),
        [pltpu.SemaphoreType.DMA] * 5,
        [pltpu.SemaphoreType.DMA] * 5,
    ],
)
def _spmm1_kernel(hs_hbm, e3_hbm, out_hbm, src_v, dst_v, bufs, acc_sh, sg, ss):
    c = lax.axis_index("c")
    s = lax.axis_index("s")
    grps = jnp.where(s < 15, 32, 20)
    pltpu.sync_copy(hs_hbm.at[c, pl.ds(s * 640, 640)], acc_sh.at[pl.ds(s * 640, 640)])
    src_h = e3_hbm.at[0]
    dst_h = e3_hbm.at[1]

    @pl.when(s < 15)
    def _stage_full():
        pltpu.sync_copy(src_h.at[pl.ds(s * 160, 160)], src_v)
        pltpu.sync_copy(dst_h.at[pl.ds(s * 160, 160)], dst_v)

    @pl.when(s == 15)
    def _stage_tail():
        pltpu.sync_copy(src_h.at[pl.ds(2400, 100)], src_v.at[pl.ds(0, 100)])
        pltpu.sync_copy(dst_h.at[pl.ds(2400, 100)], dst_v.at[pl.ds(0, 100)])

    plsc.subcore_barrier()
    hs_c = hs_hbm.at[c]

    for b in range(5):
        pltpu.async_copy(hs_c.at[src_v.at[b]], bufs.at[b], sg[b])

    def grp(qq, carry):
        j0 = qq * 5
        for b in range(5):
            pltpu.make_async_copy(hs_c.at[src_v.at[j0 + b]], bufs.at[b], sg[b]).wait()
            pltpu.async_copy(bufs.at[b], acc_sh.at[dst_v.at[j0 + b]], ss[b], add=True)

        @pl.when(qq < grps - 1)
        def _refill():
            for b in range(5):
                pltpu.make_async_copy(bufs.at[b], acc_sh.at[dst_v.at[0]], ss[b]).wait()
                pltpu.async_copy(hs_c.at[src_v.at[j0 + 5 + b]], bufs.at[b], sg[b])

        return carry

    lax.fori_loop(0, grps, grp, 0)
    for b in range(5):
        pltpu.make_async_copy(bufs.at[b], acc_sh.at[dst_v.at[0]], ss[b]).wait()
    plsc.subcore_barrier()
    pltpu.sync_copy(acc_sh.at[pl.ds(s * 640, 640)], out_hbm.at[c, pl.ds(s * 640, 640)])


@functools.partial(
    pl.kernel,
    out_type=jax.ShapeDtypeStruct((2, NP, C16), jnp.float32),
    mesh=_mesh,
    compiler_params=_sc_params,
    scratch_types=[
        pltpu.VMEM((80, 128), jnp.int32),
        pltpu.VMEM((80, 128), jnp.int32),
        pltpu.VMEM((4, 128, C16), jnp.float32),
        pltpu.VMEM_SHARED((NP, C16), jnp.float32),
        pltpu.VMEM_SHARED((NP, C16), jnp.float32),
        [pltpu.SemaphoreType.DMA] * 4,
        [pltpu.SemaphoreType.DMA] * 4,
    ],
)
def _spmm2_kernel(hs_hbm, e3_hbm, out_hbm, src_v, dst_v, bufs, hs_sh, acc_sh, sg, ss):
    c = lax.axis_index("c")
    s = lax.axis_index("s")
    wid = s * 2 + c
    quads = jnp.where(wid < 31, 20, 5)
    pltpu.sync_copy(hs_hbm.at[pl.ds(s * 640, 640)], hs_sh.at[pl.ds(s * 640, 640)])
    pltpu.sync_copy(hs_hbm.at[pl.ds(s * 640, 640)], acc_sh.at[pl.ds(s * 640, 640)])
    src_h = e3_hbm.at[0]
    dst_h = e3_hbm.at[1]

    @pl.when(wid < 31)
    def _stage_full():
        pltpu.sync_copy(src_h.at[pl.ds(wid * 80, 80)], src_v)
        pltpu.sync_copy(dst_h.at[pl.ds(wid * 80, 80)], dst_v)

    @pl.when(wid == 31)
    def _stage_tail():
        pltpu.sync_copy(src_h.at[pl.ds(2480, 20)], src_v.at[pl.ds(0, 20)])
        pltpu.sync_copy(dst_h.at[pl.ds(2480, 20)], dst_v.at[pl.ds(0, 20)])

    plsc.subcore_barrier()

    for b in range(4):
        pltpu.async_copy(hs_sh.at[src_v.at[b]], bufs.at[b], sg[b])

    def quad(qq, carry):
        j0 = qq * 4
        for b in range(4):
            pltpu.make_async_copy(hs_sh.at[src_v.at[j0 + b]], bufs.at[b], sg[b]).wait()
            pltpu.async_copy(bufs.at[b], acc_sh.at[dst_v.at[j0 + b]], ss[b], add=True)

        @pl.when(qq < quads - 1)
        def _refill():
            for b in range(4):
                pltpu.make_async_copy(bufs.at[b], acc_sh.at[dst_v.at[0]], ss[b]).wait()
                pltpu.async_copy(hs_sh.at[src_v.at[j0 + 4 + b]], bufs.at[b], sg[b])

        return carry

    lax.fori_loop(0, quads, quad, 0)
    for b in range(4):
        pltpu.make_async_copy(bufs.at[b], acc_sh.at[dst_v.at[0]], ss[b]).wait()
    plsc.subcore_barrier()
    pltpu.sync_copy(acc_sh.at[pl.ds(s * 640, 640)], out_hbm.at[c, pl.ds(s * 640, 640)])


def _k1_body(x_ref, w_ref, d0_ref, d1_ref, hs_ref, dis_ref):
    deg = d0_ref[:, 0] + d1_ref[:, 0] + 1.0
    dis = lax.rsqrt(deg)[:, None]
    dis_ref[...] = dis
    h = jnp.dot(x_ref[...], w_ref[0], preferred_element_type=jnp.float32)
    hs_ref[0] = h * dis


def _k1(x, W1r, d0, d1):
    return pl.pallas_call(
        _k1_body,
        grid=(N // BM, 2),
        in_specs=[
            pl.BlockSpec((BM, F), lambda i, j: (i, 0)),
            pl.BlockSpec((1, F, FH), lambda i, j: (j, 0, 0)),
            pl.BlockSpec((BM, 1), lambda i, j: (i, 0)),
            pl.BlockSpec((BM, 1), lambda i, j: (i, 0)),
        ],
        out_specs=[
            pl.BlockSpec((1, BM, FH), lambda i, j: (j, i, 0)),
            pl.BlockSpec((BM, 1), lambda i, j: (i, 0)),
        ],
        out_shape=[
            jax.ShapeDtypeStruct((2, NP, FH), jnp.float32),
            jax.ShapeDtypeStruct((NP, 1), jnp.float32),
        ],
    )(x, W1r, d0, d1)


def _k3_body(a0_ref, a1_ref, dis_ref, b1_ref, w2_ref, hs2_ref):
    dis = dis_ref[...]
    acc = jnp.concatenate([a0_ref[0], a1_ref[0]], axis=1)
    h1 = jnp.maximum(acc * dis + b1_ref[...], 0.0)
    hs2_ref[...] = jnp.dot(h1, w2_ref[...], preferred_element_type=jnp.float32) * dis


def _k3(acc1, disp, b1r, W2p):
    return pl.pallas_call(
        _k3_body,
        grid=(N // BM,),
        in_specs=[
            pl.BlockSpec((1, BM, FH), lambda i: (0, i, 0)),
            pl.BlockSpec((1, BM, FH), lambda i: (1, i, 0)),
            pl.BlockSpec((BM, 1), lambda i: (i, 0)),
            pl.BlockSpec((1, F), lambda i: (0, 0)),
            pl.BlockSpec((F, C16), lambda i: (0, 0)),
        ],
        out_specs=pl.BlockSpec((BM, C16), lambda i: (i, 0)),
        out_shape=jax.ShapeDtypeStruct((NP, C16), jnp.float32),
    )(acc1, acc1, disp, b1r, W2p)


def _k5_body(a0_ref, a1_ref, hs2_ref, dis_ref, b2_ref, out_ref):
    agg = a0_ref[0] + a1_ref[0] - hs2_ref[...]
    out_ref[...] = (agg * dis_ref[...] + b2_ref[...])[:, :7]


def _k5(acc2, hs2, disp, b2r):
    return pl.pallas_call(
        _k5_body,
        grid=(N // BM,),
        in_specs=[
            pl.BlockSpec((1, BM, C16), lambda i: (0, i, 0)),
            pl.BlockSpec((1, BM, C16), lambda i: (1, i, 0)),
            pl.BlockSpec((BM, C16), lambda i: (i, 0)),
            pl.BlockSpec((BM, 1), lambda i: (i, 0)),
            pl.BlockSpec((1, C16), lambda i: (0, 0)),
        ],
        out_specs=pl.BlockSpec((BM, 7), lambda i: (i, 0)),
        out_shape=jax.ShapeDtypeStruct((N, 7), jnp.float32),
    )(acc2, acc2, hs2, disp, b2r)


def kernel(x, edge_index, W1, b1, W2, b2):
    e3 = edge_index.astype(jnp.int32).reshape(2, ERV, 128)
    W1r = W1.reshape(F, 2, FH).transpose(1, 0, 2)
    W2p = jnp.pad(W2, ((0, 0), (0, C16 - W2.shape[1])))
    b1r = b1.reshape(1, F)
    b2r = jnp.pad(b2, (0, C16 - b2.shape[0])).reshape(1, C16)
    zeros_np = jnp.zeros((NP,), jnp.float32)

    deg0, deg1 = _deg_kernel(e3, zeros_np)
    hs_r, disp = _k1(x, W1r, deg0.reshape(NP, 1), deg1.reshape(NP, 1))
    acc1 = _spmm1_kernel(hs_r, e3)
    hs2 = _k3(acc1, disp, b1r, W2p)
    acc2 = _spmm2_kernel(hs2, e3)
    return _k5(acc2, hs2, disp, b2r)


# speedup vs baseline: 1.0035x; 1.0035x over previous
import functools

import jax
import jax.numpy as jnp
from jax import lax
from jax.experimental import pallas as pl
from jax.experimental.pallas import tpu as pltpu
from jax.experimental.pallas import tpu_sc as plsc

N = 10000
NP = 10240
E = 320000
ERV = E // 128
F = 128
FH = 64
C16 = 16
BM = 5000

_mesh = plsc.VectorSubcoreMesh(core_axis_name="c", subcore_axis_name="s")
_sc_params = pltpu.CompilerParams(use_tc_tiling_on_sc=False)


@functools.partial(
    pl.kernel,
    out_type=[jax.ShapeDtypeStruct((NP,), jnp.float32),
              jax.ShapeDtypeStruct((NP,), jnp.float32)],
    mesh=_mesh,
    compiler_params=_sc_params,
    scratch_types=[
        pltpu.VMEM((80, 128), jnp.int32),
        pltpu.VMEM((128,), jnp.float32),
        pltpu.VMEM_SHARED((NP,), jnp.float32),
        pltpu.SemaphoreType.DMA,
    ],
)
def _deg_kernel(e3_hbm, zeros_hbm, out0_hbm, out1_hbm, dst_v, ones_v, acc_sh, sem):
    c = lax.axis_index("c")
    s = lax.axis_index("s")
    wid = s * 2 + c
    rows = jnp.where(wid < 31, 80, 20)
    pltpu.sync_copy(zeros_hbm.at[pl.ds(s * 640, 640)], acc_sh.at[pl.ds(s * 640, 640)])
    for k in range(8):
        ones_v[pl.ds(k * 16, 16)] = jnp.full((16,), 1.0, jnp.float32)
    dst_h = e3_hbm.at[1]

    @pl.when(wid < 31)
    def _stage_full():
        pltpu.sync_copy(dst_h.at[pl.ds(wid * 80, 80)], dst_v)

    @pl.when(wid == 31)
    def _stage_tail():
        pltpu.sync_copy(dst_h.at[pl.ds(2480, 20)], dst_v.at[pl.ds(0, 20)])

    plsc.subcore_barrier()

    def fire(j, carry):
        pltpu.async_copy(ones_v, acc_sh.at[dst_v.at[j]], sem, add=True)

        @pl.when(j >= 8)
        def _bound():
            pltpu.make_async_copy(ones_v, acc_sh.at[dst_v.at[0]], sem).wait()

        return carry

    lax.fori_loop(0, rows, fire, 0)

    def drain(j, carry):
        pltpu.make_async_copy(ones_v, acc_sh.at[dst_v.at[0]], sem).wait()
        return carry

    lax.fori_loop(0, jnp.minimum(rows, 8), drain, 0)
    plsc.subcore_barrier()
    plsc.subcore_barrier()

    @pl.when(c == 0)
    def _out0():
        pltpu.sync_copy(acc_sh.at[pl.ds(s * 640, 640)], out0_hbm.at[pl.ds(s * 640, 640)])

    @pl.when(c == 1)
    def _out1():
        pltpu.sync_copy(acc_sh.at[pl.ds(s * 640, 640)], out1_hbm.at[pl.ds(s * 640, 640)])


@functools.partial(
    pl.kernel,
    out_type=jax.ShapeDtypeStruct((2, NP, FH), jnp.float32),
    mesh=_mesh,
    compiler_params=_sc_params,
    scratch_types=[
        pltpu.VMEM((160, 128), jnp.int32),
        pltpu.VMEM((160, 128), jnp.int32),
        pltpu.VMEM((5, 128, FH), jnp.float32),
        pltpu.VMEM_SHARED((NP, FH), jnp.float32),
        [pltpu.SemaphoreType.DMA] * 5,
        [pltpu.SemaphoreType.DMA] * 5,
    ],
)
def _spmm1_kernel(hs_hbm, e3_hbm, out_hbm, src_v, dst_v, bufs, acc_sh, sg, ss):
    c = lax.axis_index("c")
    s = lax.axis_index("s")
    grps = jnp.where(s < 15, 32, 20)
    pltpu.sync_copy(hs_hbm.at[c, pl.ds(s * 640, 640)], acc_sh.at[pl.ds(s * 640, 640)])
    src_h = e3_hbm.at[0]
    dst_h = e3_hbm.at[1]

    @pl.when(s < 15)
    def _stage_full():
        pltpu.sync_copy(src_h.at[pl.ds(s * 160, 160)], src_v)
        pltpu.sync_copy(dst_h.at[pl.ds(s * 160, 160)], dst_v)

    @pl.when(s == 15)
    def _stage_tail():
        pltpu.sync_copy(src_h.at[pl.ds(2400, 100)], src_v.at[pl.ds(0, 100)])
        pltpu.sync_copy(dst_h.at[pl.ds(2400, 100)], dst_v.at[pl.ds(0, 100)])

    plsc.subcore_barrier()
    hs_c = hs_hbm.at[c]

    for b in range(5):
        pltpu.async_copy(hs_c.at[src_v.at[b]], bufs.at[b], sg[b])

    def grp(qq, carry):
        j0 = qq * 5
        for b in range(5):
            pltpu.make_async_copy(hs_c.at[src_v.at[j0 + b]], bufs.at[b], sg[b]).wait()
            pltpu.async_copy(bufs.at[b], acc_sh.at[dst_v.at[j0 + b]], ss[b], add=True)

        @pl.when(qq < grps - 1)
        def _refill():
            for b in range(5):
                pltpu.make_async_copy(bufs.at[b], acc_sh.at[dst_v.at[0]], ss[b]).wait()
                pltpu.async_copy(hs_c.at[src_v.at[j0 + 5 + b]], bufs.at[b], sg[b])

        return carry

    lax.fori_loop(0, grps, grp, 0)
    for b in range(5):
        pltpu.make_async_copy(bufs.at[b], acc_sh.at[dst_v.at[0]], ss[b]).wait()
    plsc.subcore_barrier()
    plsc.subcore_barrier()
    pltpu.sync_copy(acc_sh.at[pl.ds(s * 640, 640)], out_hbm.at[c, pl.ds(s * 640, 640)])


@functools.partial(
    pl.kernel,
    out_type=jax.ShapeDtypeStruct((2, NP, C16), jnp.float32),
    mesh=_mesh,
    compiler_params=_sc_params,
    scratch_types=[
        pltpu.VMEM((80, 128), jnp.int32),
        pltpu.VMEM((80, 128), jnp.int32),
        pltpu.VMEM((4, 128, C16), jnp.float32),
        pltpu.VMEM_SHARED((NP, C16), jnp.float32),
        pltpu.VMEM_SHARED((NP, C16), jnp.float32),
        [pltpu.SemaphoreType.DMA] * 4,
        [pltpu.SemaphoreType.DMA] * 4,
    ],
)
def _spmm2_kernel(hs_hbm, e3_hbm, out_hbm, src_v, dst_v, bufs, hs_sh, acc_sh, sg, ss):
    c = lax.axis_index("c")
    s = lax.axis_index("s")
    wid = s * 2 + c
    quads = jnp.where(wid < 31, 20, 5)
    pltpu.sync_copy(hs_hbm.at[pl.ds(s * 640, 640)], hs_sh.at[pl.ds(s * 640, 640)])
    pltpu.sync_copy(hs_hbm.at[pl.ds(s * 640, 640)], acc_sh.at[pl.ds(s * 640, 640)])
    src_h = e3_hbm.at[0]
    dst_h = e3_hbm.at[1]

    @pl.when(wid < 31)
    def _stage_full():
        pltpu.sync_copy(src_h.at[pl.ds(wid * 80, 80)], src_v)
        pltpu.sync_copy(dst_h.at[pl.ds(wid * 80, 80)], dst_v)

    @pl.when(wid == 31)
    def _stage_tail():
        pltpu.sync_copy(src_h.at[pl.ds(2480, 20)], src_v.at[pl.ds(0, 20)])
        pltpu.sync_copy(dst_h.at[pl.ds(2480, 20)], dst_v.at[pl.ds(0, 20)])

    plsc.subcore_barrier()

    for b in range(4):
        pltpu.async_copy(hs_sh.at[src_v.at[b]], bufs.at[b], sg[b])

    def quad(qq, carry):
        j0 = qq * 4
        for b in range(4):
            pltpu.make_async_copy(hs_sh.at[src_v.at[j0 + b]], bufs.at[b], sg[b]).wait()
            pltpu.async_copy(bufs.at[b], acc_sh.at[dst_v.at[j0 + b]], ss[b], add=True)

        @pl.when(qq < quads - 1)
        def _refill():
            for b in range(4):
                pltpu.make_async_copy(bufs.at[b], acc_sh.at[dst_v.at[0]], ss[b]).wait()
                pltpu.async_copy(hs_sh.at[src_v.at[j0 + 4 + b]], bufs.at[b], sg[b])

        return carry

    lax.fori_loop(0, quads, quad, 0)
    for b in range(4):
        pltpu.make_async_copy(bufs.at[b], acc_sh.at[dst_v.at[0]], ss[b]).wait()
    plsc.subcore_barrier()
    plsc.subcore_barrier()
    pltpu.sync_copy(acc_sh.at[pl.ds(s * 640, 640)], out_hbm.at[c, pl.ds(s * 640, 640)])


def _k1_body(x_ref, w_ref, d0_ref, d1_ref, hs_ref, dis_ref):
    deg = d0_ref[:, 0] + d1_ref[:, 0] + 1.0
    dis = lax.rsqrt(deg)[:, None]
    dis_ref[...] = dis
    h = jnp.dot(x_ref[...], w_ref[0], preferred_element_type=jnp.float32)
    hs_ref[0] = h * dis


def _k1(x, W1r, d0, d1):
    return pl.pallas_call(
        _k1_body,
        grid=(N // BM, 2),
        in_specs=[
            pl.BlockSpec((BM, F), lambda i, j: (i, 0)),
            pl.BlockSpec((1, F, FH), lambda i, j: (j, 0, 0)),
            pl.BlockSpec((BM, 1), lambda i, j: (i, 0)),
            pl.BlockSpec((BM, 1), lambda i, j: (i, 0)),
        ],
        out_specs=[
            pl.BlockSpec((1, BM, FH), lambda i, j: (j, i, 0)),
            pl.BlockSpec((BM, 1), lambda i, j: (i, 0)),
        ],
        out_shape=[
            jax.ShapeDtypeStruct((2, NP, FH), jnp.float32),
            jax.ShapeDtypeStruct((NP, 1), jnp.float32),
        ],
    )(x, W1r, d0, d1)


def _k3_body(a0_ref, a1_ref, dis_ref, b1_ref, w2_ref, hs2_ref):
    dis = dis_ref[...]
    acc = jnp.concatenate([a0_ref[0], a1_ref[0]], axis=1)
    h1 = jnp.maximum(acc * dis + b1_ref[...], 0.0)
    hs2_ref[...] = jnp.dot(h1, w2_ref[...], preferred_element_type=jnp.float32) * dis


def _k3(acc1, disp, b1r, W2p):
    return pl.pallas_call(
        _k3_body,
        grid=(N // BM,),
        in_specs=[
            pl.BlockSpec((1, BM, FH), lambda i: (0, i, 0)),
            pl.BlockSpec((1, BM, FH), lambda i: (1, i, 0)),
            pl.BlockSpec((BM, 1), lambda i: (i, 0)),
            pl.BlockSpec((1, F), lambda i: (0, 0)),
            pl.BlockSpec((F, C16), lambda i: (0, 0)),
        ],
        out_specs=pl.BlockSpec((BM, C16), lambda i: (i, 0)),
        out_shape=jax.ShapeDtypeStruct((NP, C16), jnp.float32),
    )(acc1, acc1, disp, b1r, W2p)


def _k5_body(a0_ref, a1_ref, hs2_ref, dis_ref, b2_ref, out_ref):
    agg = a0_ref[0] + a1_ref[0] - hs2_ref[...]
    out_ref[...] = (agg * dis_ref[...] + b2_ref[...])[:, :7]


def _k5(acc2, hs2, disp, b2r):
    return pl.pallas_call(
        _k5_body,
        grid=(N // BM,),
        in_specs=[
            pl.BlockSpec((1, BM, C16), lambda i: (0, i, 0)),
            pl.BlockSpec((1, BM, C16), lambda i: (1, i, 0)),
            pl.BlockSpec((BM, C16), lambda i: (i, 0)),
            pl.BlockSpec((BM, 1), lambda i: (i, 0)),
            pl.BlockSpec((1, C16), lambda i: (0, 0)),
        ],
        out_specs=pl.BlockSpec((BM, 7), lambda i: (i, 0)),
        out_shape=jax.ShapeDtypeStruct((N, 7), jnp.float32),
    )(acc2, acc2, hs2, disp, b2r)


def kernel(x, edge_index, W1, b1, W2, b2):
    e3 = edge_index.astype(jnp.int32).reshape(2, ERV, 128)
    W1r = W1.reshape(F, 2, FH).transpose(1, 0, 2)
    W2p = jnp.pad(W2, ((0, 0), (0, C16 - W2.shape[1])))
    b1r = b1.reshape(1, F)
    b2r = jnp.pad(b2, (0, C16 - b2.shape[0])).reshape(1, C16)
    zeros_np = jnp.zeros((NP,), jnp.float32)

    deg0, deg1 = _deg_kernel(e3, zeros_np)
    hs_r, disp = _k1(x, W1r, deg0.reshape(NP, 1), deg1.reshape(NP, 1))
    acc1 = _spmm1_kernel(hs_r, e3)
    hs2 = _k3(acc1, disp, b1r, W2p)
    acc2 = _spmm2_kernel(hs2, e3)
    return _k5(acc2, hs2, disp, b2r)
